# baseline (speedup 1.0000x reference)
import functools

import jax
import jax.numpy as jnp
from jax import lax
from jax.experimental import pallas as pl
from jax.experimental.pallas import tpu as pltpu
from jax.experimental.pallas import tpu_sc as plsc

NC = 2
NS = 16
CH = 64
NBUF = 4
ED_AHEAD = 3
G_AHEAD = 2
RB = 400


def _gather16(v16, idx):
  dnums = lax.GatherDimensionNumbers(
      offset_dims=(), collapsed_slice_dims=(0,), start_index_map=(0,))
  return lax.gather(v16, idx[:, None], dnums, slice_sizes=(1,),
                    mode=lax.GatherScatterMode.PROMISE_IN_BOUNDS)


def _bcast_lane(v16, j):
  return _gather16(v16, jnp.full((16,), j, dtype=jnp.int32))


def _prefix16(m_i32, lane):
  v = m_i32
  for sh in (1, 2, 4, 8):
    shifted = _gather16(v, jnp.maximum(lane - sh, 0))
    v = v + jnp.where(lane >= sh, shifted, 0)
  return v


def _make_spmm(n_rows, n_tab, D, EPW, col_split, tab_in_spmem=False,
               NBUF=NBUF, ED_AHEAD=ED_AHEAD, G_AHEAD=G_AHEAD,
               do_scatter=True):
  NCHUNK = EPW // CH
  RPSF = (-(-n_rows // NS) + 7) // 8 * 8
  RPSL = n_rows - (NS - 1) * RPSF
  assert RPSL > 0
  G16 = CH // 16
  DV = D // 16
  mesh = plsc.VectorSubcoreMesh(core_axis_name="c", subcore_axis_name="s",
                                num_cores=NC, num_subcores=NS)

  assert NCHUNK % NBUF == 0 and NCHUNK >= NBUF + ED_AHEAD

  @functools.partial(
      pl.kernel,
      out_type=jax.ShapeDtypeStruct((2 * n_rows, D), jnp.float32),
      mesh=mesh,
      scratch_types=[
          pltpu.VMEM_SHARED((n_rows, D), jnp.float32),
      ]
      + ([pltpu.VMEM_SHARED((n_tab, D), jnp.float32)] if tab_in_spmem else [])
      + [pltpu.VMEM((CH, D), jnp.float32) for _ in range(NBUF)]
      + [pltpu.VMEM((CH,), jnp.int32) for _ in range(NBUF)]
      + [pltpu.VMEM((CH,), jnp.int32) for _ in range(NBUF)]
      + [pltpu.VMEM((CH,), jnp.float32) for _ in range(NBUF)]
      + [pltpu.SemaphoreType.DMA for _ in range(3 * NBUF + 2)],
  )
  def spmm(tab_hbm, src_hbm, dst_hbm, w_hbm, zer_hbm, out_hbm,
           accum, *bufs):
    if tab_in_spmem:
      tabs = bufs[0]
      bufs = bufs[1:]
    rows = bufs[:NBUF]
    srcc = bufs[NBUF:2 * NBUF]
    dstc = bufs[2 * NBUF:3 * NBUF]
    wc = bufs[3 * NBUF:4 * NBUF]
    esem = bufs[4 * NBUF:5 * NBUF]
    gsem = bufs[5 * NBUF:6 * NBUF]
    ssem = bufs[6 * NBUF:7 * NBUF]
    zsem = bufs[7 * NBUF]
    tsem = bufs[7 * NBUF + 1]
    c = lax.axis_index("c")
    s = lax.axis_index("s")
    if col_split:
      ebase_src = (c * NS + s) * EPW
      ebase_dw = s * EPW
    else:
      sl = s * NC + c
      ebase_src = sl * EPW
      ebase_dw = sl * EPW
    ebase_src = pl.multiple_of(ebase_src, 8)
    ebase_dw = pl.multiple_of(ebase_dw, 8)
    rbase = pl.multiple_of(s * RPSF, 8)

    @pl.when(s < NS - 1)
    def _():
      pltpu.async_copy(zer_hbm, accum.at[pl.ds(rbase, RPSF)], zsem).wait()

    @pl.when(s == NS - 1)
    def _():
      pltpu.async_copy(zer_hbm.at[pl.ds(0, RPSL)],
                       accum.at[pl.ds(rbase, RPSL)], zsem).wait()

    if tab_in_spmem:
      TPSF = (-(-n_tab // NS) + 7) // 8 * 8
      TPSL = n_tab - (NS - 1) * TPSF
      assert TPSL > 0
      tbase = pl.multiple_of(s * TPSF, 8)

      @pl.when(s < NS - 1)
      def _():
        pltpu.async_copy(tab_hbm.at[pl.ds(tbase, TPSF)],
                         tabs.at[pl.ds(tbase, TPSF)], tsem).wait()

      @pl.when(s == NS - 1)
      def _():
        pltpu.async_copy(tab_hbm.at[pl.ds(tbase, TPSL)],
                         tabs.at[pl.ds(tbase, TPSL)], tsem).wait()

    plsc.subcore_barrier()

    def start_edges(g, b):
      gb = pl.multiple_of(g * CH, CH)
      pltpu.async_copy(src_hbm.at[pl.ds(ebase_src + gb, CH)], srcc[b], esem[b])
      pltpu.async_copy(dst_hbm.at[pl.ds(ebase_dw + gb, CH)], dstc[b], esem[b])
      pltpu.async_copy(w_hbm.at[pl.ds(ebase_dw + gb, CH)], wc[b], esem[b])

    def wait_edges(b):
      pltpu.make_async_copy(src_hbm.at[pl.ds(0, CH)], srcc[b], esem[b]).wait()
      pltpu.make_async_copy(dst_hbm.at[pl.ds(0, CH)], dstc[b], esem[b]).wait()
      pltpu.make_async_copy(w_hbm.at[pl.ds(0, CH)], wc[b], esem[b]).wait()

    tab_ref = tabs if tab_in_spmem else tab_hbm

    def start_gather(b):
      pltpu.async_copy(tab_ref.at[srcc[b]], rows[b], gsem[b])

    def wait_gather(b):
      pltpu.make_async_copy(tab_ref.at[srcc[b]], rows[b], gsem[b]).wait()

    def wait_scatter(b):
      pltpu.make_async_copy(rows[b], accum.at[dstc[b]], ssem[b]).wait()

    for k in range(ED_AHEAD):
      start_edges(k, k)
    for k in range(G_AHEAD):
      wait_edges(k)
      start_gather(k)

    def outer(gg, carry):
      for b in range(NBUF):
        g = gg * NBUF + b
        se = (b + ED_AHEAD) % NBUF
        sg = (b + G_AHEAD) % NBUF
        if do_scatter:
          @pl.when(jnp.logical_and(g >= NBUF - ED_AHEAD,
                                   g + ED_AHEAD < NCHUNK))
          def _():
            wait_scatter(se)

        @pl.when(g + ED_AHEAD < NCHUNK)
        def _():
          start_edges(g + ED_AHEAD, se)

        @pl.when(g + G_AHEAD < NCHUNK)
        def _():
          wait_edges(sg)
          start_gather(sg)

        wait_gather(b)
        for grp in range(G16):
          w16 = wc[b][pl.ds(grp * 16, 16)]
          for j in range(16):
            wj = _bcast_lane(w16, j)
            e = grp * 16 + j
            for k in range(DV):
              csl = pl.ds(k * 16, 16)
              rows[b][e, csl] = rows[b][e, csl] * wj
        if do_scatter:
          pltpu.async_copy(rows[b], accum.at[dstc[b]], ssem[b], add=True)
      return carry

    lax.fori_loop(0, NCHUNK // NBUF, outer, 0)
    if do_scatter:
      for b in range(NBUF):
        wait_scatter(b)
    plsc.subcore_barrier()

    obase = pl.multiple_of(c * n_rows + rbase, 8)

    @pl.when(s < NS - 1)
    def _():
      pltpu.sync_copy(accum.at[pl.ds(rbase, RPSF)],
                      out_hbm.at[pl.ds(obase, RPSF)])

    @pl.when(s == NS - 1)
    def _():
      pltpu.sync_copy(accum.at[pl.ds(rbase, RPSL)],
                      out_hbm.at[pl.ds(obase, RPSL)])

  return spmm


CH1 = 32
NBUF1 = 3
ED1 = 2
GA1 = 1
PADQ = CH1


def _make_part(ep, n):
  EPP = ep // (2 * NS)
  EPPL = -(-EPP // PADQ) * PADQ
  VBUF = EPPL + PADQ + 16
  HALF = n // 2
  NG = EPP // 16
  mesh = plsc.VectorSubcoreMesh(core_axis_name="c", subcore_axis_name="s",
                                num_cores=NC, num_subcores=NS)

  @functools.partial(
      pl.kernel,
      out_type=[
          jax.ShapeDtypeStruct((2 * 2 * NS * EPPL,), jnp.int32),
          jax.ShapeDtypeStruct((2 * 2 * NS * EPPL,), jnp.int32),
          jax.ShapeDtypeStruct((2 * 2 * NS * EPPL,), jnp.float32),
          jax.ShapeDtypeStruct((2 * NS * 8,), jnp.int32),
      ],
      mesh=mesh,
      scratch_types=[
          pltpu.VMEM((EPP,), jnp.int32),
          pltpu.VMEM((EPP,), jnp.int32),
          pltpu.VMEM((EPP,), jnp.float32),
          pltpu.VMEM((VBUF,), jnp.int32),
          pltpu.VMEM((VBUF,), jnp.int32),
          pltpu.VMEM((VBUF,), jnp.float32),
          pltpu.VMEM((VBUF,), jnp.int32),
          pltpu.VMEM((VBUF,), jnp.int32),
          pltpu.VMEM((VBUF,), jnp.float32),
          pltpu.VMEM((16,), jnp.int32),
      ],
  )
  def part(src_hbm, dst_hbm, w_hbm, ps_hbm, pd_hbm, pw_hbm, cnt_hbm,
           sv, dv, wv, oas, oad, oaw, obs, obd, obw, cntv):
    c = lax.axis_index("c")
    s = lax.axis_index("s")
    wid = s * NC + c
    ebase = pl.multiple_of(wid * EPP, 8)
    pltpu.sync_copy(src_hbm.at[pl.ds(ebase, EPP)], sv)
    pltpu.sync_copy(dst_hbm.at[pl.ds(ebase, EPP)], dv)
    pltpu.sync_copy(w_hbm.at[pl.ds(ebase, EPP)], wv)

    def grp(i, carry):
      na, nb = carry
      lane = lax.iota(jnp.int32, 16)
      o = pl.ds(pl.multiple_of(i * 16, 16), 16)
      s16 = sv[o]
      d16 = dv[o]
      w16 = wv[o]
      ma = d16 < HALF
      cma = _prefix16(jnp.where(ma, 1, 0), lane)
      ca = cma[15]
      cav = _bcast_lane(cma, 15)
      pos = jnp.where(ma, cma - 1, cav + (lane + 1) - cma - 1)
      perm = jnp.zeros((16,), jnp.int32)
      for j in range(16):
        perm = perm + jnp.where(_bcast_lane(pos, j) == lane, j, 0)
      oas[pl.ds(na, 16)] = _gather16(s16, perm)
      oad[pl.ds(na, 16)] = _gather16(d16, perm)
      oaw[pl.ds(na, 16)] = _gather16(w16, perm)
      permb = _gather16(perm, jnp.minimum(lane + cav, 15))
      obs[pl.ds(nb, 16)] = _gather16(s16, permb)
      obd[pl.ds(nb, 16)] = _gather16(d16, permb) - HALF
      obw[pl.ds(nb, 16)] = _gather16(w16, permb)
      return na + ca, nb + (16 - ca)

    na, nb = lax.fori_loop(0, NG, grp, (0, 0))
    lane = lax.iota(jnp.int32, 16)
    zi = jnp.zeros((16,), jnp.int32)
    zf = jnp.zeros((16,), jnp.float32)
    for k in range(PADQ // 16):
      oas[pl.ds(na + k * 16, 16)] = zi
      oad[pl.ds(na + k * 16, 16)] = zi
      oaw[pl.ds(na + k * 16, 16)] = zf
      obs[pl.ds(nb + k * 16, 16)] = zi
      obd[pl.ds(nb + k * 16, 16)] = zi
      obw[pl.ds(nb + k * 16, 16)] = zf
    napv = jnp.bitwise_and(na + (PADQ - 1), -PADQ)
    nbpv = jnp.bitwise_and(nb + (PADQ - 1), -PADQ)

    cntv[pl.ds(0, 16)] = jnp.where(
        lane == 0, napv, jnp.where(lane == 1, nbpv, 0))
    pltpu.sync_copy(cntv.at[pl.ds(0, 8)], cnt_hbm.at[pl.ds(wid * 8, 8)])

  return part


def _make_spmm_dst(n, EPP):
  HALF = n // 2
  EPPL = -(-EPP // PADQ) * PADQ
  TRIPS = EPPL // (CH1 * NBUF1)
  RPSF = (-(-HALF // NS) + 7) // 8 * 8
  RPSL = HALF - (NS - 1) * RPSF
  assert RPSL > 0
  mesh = plsc.VectorSubcoreMesh(core_axis_name="c", subcore_axis_name="s",
                                num_cores=NC, num_subcores=NS)

  @functools.partial(
      pl.kernel,
      out_type=[
          jax.ShapeDtypeStruct((n, 128), jnp.float32),
          jax.ShapeDtypeStruct((n, 128), jnp.float32),
      ],
      mesh=mesh,
      scratch_types=[
          pltpu.VMEM_SHARED((HALF, 128), jnp.float32),
          pltpu.VMEM_SHARED((HALF, 128), jnp.float32),
      ]
      + [pltpu.VMEM((CH1, 256), jnp.float32) for _ in range(NBUF1)]
      + [pltpu.VMEM((CH1, 128), jnp.float32) for _ in range(NBUF1)]
      + [pltpu.VMEM((CH1, 128), jnp.float32) for _ in range(NBUF1)]
      + [pltpu.VMEM((CH1,), jnp.int32) for _ in range(NBUF1)]
      + [pltpu.VMEM((CH1,), jnp.int32) for _ in range(NBUF1)]
      + [pltpu.VMEM((CH1,), jnp.float32) for _ in range(NBUF1)]
      + [pltpu.VMEM((16,), jnp.int32)]
      + [pltpu.SemaphoreType.DMA for _ in range(4 * NBUF1 + 1)],
  )
  def spmm1(tab_hbm, ps_hbm, pd_hbm, pw_hbm, cnt_hbm, zer_hbm,
            ha_hbm, hb_hbm, accl, accr, *bufs):
    rows = bufs[:NBUF1]
    rwl = bufs[NBUF1:2 * NBUF1]
    rwr = bufs[2 * NBUF1:3 * NBUF1]
    srcc = bufs[3 * NBUF1:4 * NBUF1]
    dstc = bufs[4 * NBUF1:5 * NBUF1]
    wc = bufs[5 * NBUF1:6 * NBUF1]
    cntv = bufs[6 * NBUF1]
    esem = bufs[6 * NBUF1 + 1:7 * NBUF1 + 1]
    gsem = bufs[7 * NBUF1 + 1:8 * NBUF1 + 1]
    sseml = bufs[8 * NBUF1 + 1:9 * NBUF1 + 1]
    ssemr = bufs[9 * NBUF1 + 1:10 * NBUF1 + 1]
    zsem = bufs[10 * NBUF1 + 1]
    c = lax.axis_index("c")
    s = lax.axis_index("s")
    rbase = pl.multiple_of(s * RPSF, 8)

    @pl.when(s < NS - 1)
    def _():
      pltpu.async_copy(zer_hbm.at[pl.ds(0, RPSF)],
                       accl.at[pl.ds(rbase, RPSF)], zsem).wait()
      pltpu.async_copy(zer_hbm.at[pl.ds(0, RPSF)],
                       accr.at[pl.ds(rbase, RPSF)], zsem).wait()

    @pl.when(s == NS - 1)
    def _():
      pltpu.async_copy(zer_hbm.at[pl.ds(0, RPSL)],
                       accl.at[pl.ds(rbase, RPSL)], zsem).wait()
      pltpu.async_copy(zer_hbm.at[pl.ds(0, RPSL)],
                       accr.at[pl.ds(rbase, RPSL)], zsem).wait()

    plsc.subcore_barrier()

    for sub in range(2):
      wid2 = 2 * s + sub
      pltpu.sync_copy(cnt_hbm.at[pl.ds(wid2 * 8, 8)], cntv.at[pl.ds(0, 8)])
      cv = cntv[pl.ds(0, 16)]
      cnt = jnp.where(c == 0, cv[0], cv[1])
      lbase = pl.multiple_of((c * 2 * NS + wid2) * EPPL, 8)

      def start_edges(g, b):
        gb = pl.multiple_of(lbase + g * CH1, 8)
        pltpu.async_copy(ps_hbm.at[pl.ds(gb, CH1)], srcc[b], esem[b])
        pltpu.async_copy(pd_hbm.at[pl.ds(gb, CH1)], dstc[b], esem[b])
        pltpu.async_copy(pw_hbm.at[pl.ds(gb, CH1)], wc[b], esem[b])

      def wait_edges(b):
        pltpu.make_async_copy(ps_hbm.at[pl.ds(0, CH1)], srcc[b],
                              esem[b]).wait()
        pltpu.make_async_copy(pd_hbm.at[pl.ds(0, CH1)], dstc[b],
                              esem[b]).wait()
        pltpu.make_async_copy(pw_hbm.at[pl.ds(0, CH1)], wc[b],
                              esem[b]).wait()

      def start_gather(b):
        pltpu.async_copy(tab_hbm.at[srcc[b]], rows[b], gsem[b])

      def wait_gather(b):
        pltpu.make_async_copy(tab_hbm.at[srcc[b]], rows[b], gsem[b]).wait()

      def wait_scatters(b):
        pltpu.make_async_copy(rwl[b], accl.at[dstc[b]], sseml[b]).wait()
        pltpu.make_async_copy(rwr[b], accr.at[dstc[b]], ssemr[b]).wait()

      for k in range(ED1):
        @pl.when(k * CH1 < cnt)
        def _(k=k):
          start_edges(k, k)
      for k in range(GA1):
        @pl.when(k * CH1 < cnt)
        def _(k=k):
          wait_edges(k)
          start_gather(k)

      def outer(gg, carry):
        for b in range(NBUF1):
          g = gg * NBUF1 + b
          se = (b + ED1) % NBUF1
          sg = (b + GA1) % NBUF1

          @pl.when(jnp.logical_and(g >= NBUF1 - ED1,
                                   (g + ED1) * CH1 < cnt))
          def _():
            wait_scatters(se)

          @pl.when((g + ED1) * CH1 < cnt)
          def _():
            start_edges(g + ED1, se)

          @pl.when((g + GA1) * CH1 < cnt)
          def _():
            wait_edges(sg)
            start_gather(sg)

          live = g * CH1 < cnt

          @pl.when(live)
          def _():
            wait_gather(b)
            for grp in range(CH1 // 16):
              w16 = wc[b][pl.ds(grp * 16, 16)]
              for j in range(16):
                wj = _bcast_lane(w16, j)
                e = grp * 16 + j
                for k in range(16):
                  v = rows[b][e, pl.ds(k * 16, 16)] * wj
                  if k < 8:
                    rwl[b][e, pl.ds(k * 16, 16)] = v
                  else:
                    rwr[b][e, pl.ds((k - 8) * 16, 16)] = v
            pltpu.async_copy(rwl[b], accl.at[dstc[b]], sseml[b], add=True)
            pltpu.async_copy(rwr[b], accr.at[dstc[b]], ssemr[b], add=True)
        return carry

      lax.fori_loop(0, TRIPS, outer, 0)
      for b in range(NBUF1):
        @pl.when(jnp.logical_or(cnt >= NBUF1 * CH1, b * CH1 < cnt))
        def _(b=b):
          wait_scatters(b)

    plsc.subcore_barrier()
    obase = pl.multiple_of(c * HALF + rbase, 8)

    @pl.when(s < NS - 1)
    def _():
      pltpu.sync_copy(accl.at[pl.ds(rbase, RPSF)],
                      ha_hbm.at[pl.ds(obase, RPSF)])
      pltpu.sync_copy(accr.at[pl.ds(rbase, RPSF)],
                      hb_hbm.at[pl.ds(obase, RPSF)])

    @pl.when(s == NS - 1)
    def _():
      pltpu.sync_copy(accl.at[pl.ds(rbase, RPSL)],
                      ha_hbm.at[pl.ds(obase, RPSL)])
      pltpu.sync_copy(accr.at[pl.ds(rbase, RPSL)],
                      hb_hbm.at[pl.ds(obase, RPSL)])

  return spmm1


def _mm1(x, W1, n):
  nb = n // RB

  def body(x_ref, w_ref, o_ref):
    o_ref[...] = jnp.dot(x_ref[...], w_ref[...],
                         preferred_element_type=jnp.float32)

  return pl.pallas_call(
      body,
      grid=(NC, nb),
      in_specs=[
          pl.BlockSpec((RB, x.shape[1]), lambda c, i: (i, 0)),
          pl.BlockSpec((W1.shape[0], 128), lambda c, i: (0, c)),
      ],
      out_specs=pl.BlockSpec((RB, 128), lambda c, i, _nb=nb: (c * _nb + i, 0)),
      out_shape=jax.ShapeDtypeStruct((2 * n, 128), jnp.float32),
  )(x, W1)


def _mm2(h1, W2, n):
  nb = n // RB

  def body(a_ref, b_ref, w_ref, o_ref):
    w = w_ref[...]
    a = jnp.maximum(a_ref[...], 0.0)
    b = jnp.maximum(b_ref[...], 0.0)
    o_ref[...] = (
        jnp.dot(a, w[:128], preferred_element_type=jnp.float32)
        + jnp.dot(b, w[128:], preferred_element_type=jnp.float32))

  return pl.pallas_call(
      body,
      grid=(nb,),
      in_specs=[
          pl.BlockSpec((RB, 128), lambda i: (i, 0)),
          pl.BlockSpec((RB, 128), lambda i, _nb=nb: (_nb + i, 0)),
          pl.BlockSpec(W2.shape, lambda i: (0, 0)),
      ],
      out_specs=pl.BlockSpec((RB, W2.shape[1]), lambda i: (i, 0)),
      out_shape=jax.ShapeDtypeStruct((n, W2.shape[1]), jnp.float32),
  )(h1, h1, W2)


def _finish(p, n, ncls):
  nb = n // RB

  def body(a_ref, b_ref, o_ref):
    z = jnp.maximum(a_ref[:, :ncls] + b_ref[:, :ncls], 0.0)
    z = z - jnp.max(z, axis=1, keepdims=True)
    o_ref[...] = z - jnp.log(jnp.sum(jnp.exp(z), axis=1, keepdims=True))

  return pl.pallas_call(
      body,
      grid=(nb,),
      in_specs=[
          pl.BlockSpec((RB, 128), lambda i: (i, 0)),
          pl.BlockSpec((RB, 128), lambda i, _nb=nb: (_nb + i, 0)),
      ],
      out_specs=pl.BlockSpec((RB, ncls), lambda i: (i, 0)),
      out_shape=jax.ShapeDtypeStruct((n, ncls), jnp.float32),
  )(p, p)


@jax.jit
def kernel(x, edge_index, edge_weight, W1, W2):
  n = x.shape[0]
  ncls = W2.shape[1]
  e = edge_weight.shape[0]

  quant = NC * NS * CH * NBUF
  ep = ((e + quant - 1) // quant) * quant
  pad = ep - e
  src = jnp.concatenate([edge_index[1], jnp.zeros((pad,), jnp.int32)])
  dst = jnp.concatenate([edge_index[0], jnp.zeros((pad,), jnp.int32)])
  w = jnp.concatenate([edge_weight, jnp.zeros((pad,), jnp.float32)])

  rpsf = (-(-n // NS) + 7) // 8 * 8
  zer = jnp.zeros((rpsf, 128), jnp.float32)
  W2p = jnp.pad(W2, ((0, 0), (0, 128 - ncls)))

  src2 = jnp.concatenate([src, src + n])
  sup1 = _mm1(x, W1, n)
  spmm1 = _make_spmm(n, 2 * n, 128, ep // NS, col_split=True)
  h1 = spmm1(sup1, src2, dst, w, zer)
  sup2 = _mm2(h1, W2p, n)
  spmm2 = _make_spmm(n, n, 128, ep // (NC * NS), col_split=False)
  p = spmm2(sup2, src, dst, w, zer)
  return _finish(p, n, ncls)


# speedup vs baseline: 1.0365x; 1.0365x over previous
import functools

import jax
import jax.numpy as jnp
from jax import lax
from jax.experimental import pallas as pl
from jax.experimental.pallas import tpu as pltpu
from jax.experimental.pallas import tpu_sc as plsc

NC = 2
NS = 16
CH = 32
NBUF = 8
ED_AHEAD = 6
G_AHEAD = 4
RB = 400


def _gather16(v16, idx):
  dnums = lax.GatherDimensionNumbers(
      offset_dims=(), collapsed_slice_dims=(0,), start_index_map=(0,))
  return lax.gather(v16, idx[:, None], dnums, slice_sizes=(1,),
                    mode=lax.GatherScatterMode.PROMISE_IN_BOUNDS)


def _bcast_lane(v16, j):
  return _gather16(v16, jnp.full((16,), j, dtype=jnp.int32))


def _make_spmm(n_rows, n_tab, D, EPW, col_split, tab_in_spmem=False,
               NBUF=NBUF, ED_AHEAD=ED_AHEAD, G_AHEAD=G_AHEAD,
               do_scatter=True):
  NCHUNK = EPW // CH
  RPSF = (-(-n_rows // NS) + 7) // 8 * 8
  RPSL = n_rows - (NS - 1) * RPSF
  assert RPSL > 0
  G16 = CH // 16
  DV = D // 16
  mesh = plsc.VectorSubcoreMesh(core_axis_name="c", subcore_axis_name="s",
                                num_cores=NC, num_subcores=NS)

  assert NCHUNK % NBUF == 0 and NCHUNK >= NBUF + ED_AHEAD

  @functools.partial(
      pl.kernel,
      out_type=jax.ShapeDtypeStruct((2 * n_rows, D), jnp.float32),
      mesh=mesh,
      scratch_types=[
          pltpu.VMEM_SHARED((n_rows, D), jnp.float32),
      ]
      + ([pltpu.VMEM_SHARED((n_tab, D), jnp.float32)] if tab_in_spmem else [])
      + [pltpu.VMEM((CH, D), jnp.float32) for _ in range(NBUF)]
      + [pltpu.VMEM((CH,), jnp.int32) for _ in range(NBUF)]
      + [pltpu.VMEM((CH,), jnp.int32) for _ in range(NBUF)]
      + [pltpu.VMEM((CH,), jnp.float32) for _ in range(NBUF)]
      + [pltpu.SemaphoreType.DMA for _ in range(3 * NBUF + 2)],
  )
  def spmm(tab_hbm, src_hbm, dst_hbm, w_hbm, zer_hbm, out_hbm,
           accum, *bufs):
    if tab_in_spmem:
      tabs = bufs[0]
      bufs = bufs[1:]
    rows = bufs[:NBUF]
    srcc = bufs[NBUF:2 * NBUF]
    dstc = bufs[2 * NBUF:3 * NBUF]
    wc = bufs[3 * NBUF:4 * NBUF]
    esem = bufs[4 * NBUF:5 * NBUF]
    gsem = bufs[5 * NBUF:6 * NBUF]
    ssem = bufs[6 * NBUF:7 * NBUF]
    zsem = bufs[7 * NBUF]
    tsem = bufs[7 * NBUF + 1]
    c = lax.axis_index("c")
    s = lax.axis_index("s")
    if col_split:
      ebase_src = (c * NS + s) * EPW
      ebase_dw = s * EPW
    else:
      sl = s * NC + c
      ebase_src = sl * EPW
      ebase_dw = sl * EPW
    ebase_src = pl.multiple_of(ebase_src, 8)
    ebase_dw = pl.multiple_of(ebase_dw, 8)
    rbase = pl.multiple_of(s * RPSF, 8)

    @pl.when(s < NS - 1)
    def _():
      pltpu.async_copy(zer_hbm, accum.at[pl.ds(rbase, RPSF)], zsem).wait()

    @pl.when(s == NS - 1)
    def _():
      pltpu.async_copy(zer_hbm.at[pl.ds(0, RPSL)],
                       accum.at[pl.ds(rbase, RPSL)], zsem).wait()

    if tab_in_spmem:
      TPSF = (-(-n_tab // NS) + 7) // 8 * 8
      TPSL = n_tab - (NS - 1) * TPSF
      assert TPSL > 0
      tbase = pl.multiple_of(s * TPSF, 8)

      @pl.when(s < NS - 1)
      def _():
        pltpu.async_copy(tab_hbm.at[pl.ds(tbase, TPSF)],
                         tabs.at[pl.ds(tbase, TPSF)], tsem).wait()

      @pl.when(s == NS - 1)
      def _():
        pltpu.async_copy(tab_hbm.at[pl.ds(tbase, TPSL)],
                         tabs.at[pl.ds(tbase, TPSL)], tsem).wait()

    plsc.subcore_barrier()

    def start_edges(g, b):
      gb = pl.multiple_of(g * CH, CH)
      pltpu.async_copy(src_hbm.at[pl.ds(ebase_src + gb, CH)], srcc[b], esem[b])
      pltpu.async_copy(dst_hbm.at[pl.ds(ebase_dw + gb, CH)], dstc[b], esem[b])
      pltpu.async_copy(w_hbm.at[pl.ds(ebase_dw + gb, CH)], wc[b], esem[b])

    def wait_edges(b):
      pltpu.make_async_copy(src_hbm.at[pl.ds(0, CH)], srcc[b], esem[b]).wait()
      pltpu.make_async_copy(dst_hbm.at[pl.ds(0, CH)], dstc[b], esem[b]).wait()
      pltpu.make_async_copy(w_hbm.at[pl.ds(0, CH)], wc[b], esem[b]).wait()

    tab_ref = tabs if tab_in_spmem else tab_hbm

    def start_gather(b):
      pltpu.async_copy(tab_ref.at[srcc[b]], rows[b], gsem[b])

    def wait_gather(b):
      pltpu.make_async_copy(tab_ref.at[srcc[b]], rows[b], gsem[b]).wait()

    def wait_scatter(b):
      pltpu.make_async_copy(rows[b], accum.at[dstc[b]], ssem[b]).wait()

    for k in range(ED_AHEAD):
      start_edges(k, k)
    for k in range(G_AHEAD):
      wait_edges(k)
      start_gather(k)

    def outer(gg, carry):
      for b in range(NBUF):
        g = gg * NBUF + b
        se = (b + ED_AHEAD) % NBUF
        sg = (b + G_AHEAD) % NBUF
        if do_scatter:
          @pl.when(jnp.logical_and(g >= NBUF - ED_AHEAD,
                                   g + ED_AHEAD < NCHUNK))
          def _():
            wait_scatter(se)

        @pl.when(g + ED_AHEAD < NCHUNK)
        def _():
          start_edges(g + ED_AHEAD, se)

        @pl.when(g + G_AHEAD < NCHUNK)
        def _():
          wait_edges(sg)
          start_gather(sg)

        wait_gather(b)
        for grp in range(G16):
          w16 = wc[b][pl.ds(grp * 16, 16)]
          for j in range(16):
            wj = _bcast_lane(w16, j)
            e = grp * 16 + j
            for k in range(DV):
              csl = pl.ds(k * 16, 16)
              rows[b][e, csl] = rows[b][e, csl] * wj
        if do_scatter:
          pltpu.async_copy(rows[b], accum.at[dstc[b]], ssem[b], add=True)
      return carry

    lax.fori_loop(0, NCHUNK // NBUF, outer, 0)
    if do_scatter:
      for b in range(NBUF):
        wait_scatter(b)
    plsc.subcore_barrier()

    obase = pl.multiple_of(c * n_rows + rbase, 8)

    @pl.when(s < NS - 1)
    def _():
      pltpu.sync_copy(accum.at[pl.ds(rbase, RPSF)],
                      out_hbm.at[pl.ds(obase, RPSF)])

    @pl.when(s == NS - 1)
    def _():
      pltpu.sync_copy(accum.at[pl.ds(rbase, RPSL)],
                      out_hbm.at[pl.ds(obase, RPSL)])

  return spmm


def _mm1(x, W1, n):
  nb = n // RB

  def body(x_ref, w_ref, o_ref):
    o_ref[...] = jnp.dot(x_ref[...], w_ref[...],
                         preferred_element_type=jnp.float32)

  return pl.pallas_call(
      body,
      grid=(NC, nb),
      in_specs=[
          pl.BlockSpec((RB, x.shape[1]), lambda c, i: (i, 0)),
          pl.BlockSpec((W1.shape[0], 128), lambda c, i: (0, c)),
      ],
      out_specs=pl.BlockSpec((RB, 128), lambda c, i, _nb=nb: (c * _nb + i, 0)),
      out_shape=jax.ShapeDtypeStruct((2 * n, 128), jnp.float32),
  )(x, W1)


def _mm2(h1, W2, n):
  nb = n // RB

  def body(a_ref, b_ref, w_ref, o_ref):
    w = w_ref[...]
    a = jnp.maximum(a_ref[...], 0.0)
    b = jnp.maximum(b_ref[...], 0.0)
    o_ref[...] = (
        jnp.dot(a, w[:128], preferred_element_type=jnp.float32)
        + jnp.dot(b, w[128:], preferred_element_type=jnp.float32))

  return pl.pallas_call(
      body,
      grid=(nb,),
      in_specs=[
          pl.BlockSpec((RB, 128), lambda i: (i, 0)),
          pl.BlockSpec((RB, 128), lambda i, _nb=nb: (_nb + i, 0)),
          pl.BlockSpec(W2.shape, lambda i: (0, 0)),
      ],
      out_specs=pl.BlockSpec((RB, W2.shape[1]), lambda i: (i, 0)),
      out_shape=jax.ShapeDtypeStruct((n, W2.shape[1]), jnp.float32),
  )(h1, h1, W2)


def _finish(p, n, ncls):
  nb = n // RB

  def body(a_ref, b_ref, o_ref):
    z = jnp.maximum(a_ref[:, :ncls] + b_ref[:, :ncls], 0.0)
    z = z - jnp.max(z, axis=1, keepdims=True)
    o_ref[...] = z - jnp.log(jnp.sum(jnp.exp(z), axis=1, keepdims=True))

  return pl.pallas_call(
      body,
      grid=(nb,),
      in_specs=[
          pl.BlockSpec((RB, 128), lambda i: (i, 0)),
          pl.BlockSpec((RB, 128), lambda i, _nb=nb: (_nb + i, 0)),
      ],
      out_specs=pl.BlockSpec((RB, ncls), lambda i: (i, 0)),
      out_shape=jax.ShapeDtypeStruct((n, ncls), jnp.float32),
  )(p, p)


@jax.jit
def kernel(x, edge_index, edge_weight, W1, W2):
  n = x.shape[0]
  ncls = W2.shape[1]
  e = edge_weight.shape[0]

  quant = NC * NS * CH * NBUF
  ep = ((e + quant - 1) // quant) * quant
  pad = ep - e
  src = jnp.concatenate([edge_index[1], jnp.zeros((pad,), jnp.int32)])
  dst = jnp.concatenate([edge_index[0], jnp.zeros((pad,), jnp.int32)])
  w = jnp.concatenate([edge_weight, jnp.zeros((pad,), jnp.float32)])

  rpsf = (-(-n // NS) + 7) // 8 * 8
  zer = jnp.zeros((rpsf, 128), jnp.float32)
  W2p = jnp.pad(W2, ((0, 0), (0, 128 - ncls)))

  src2 = jnp.concatenate([src, src + n])
  sup1 = _mm1(x, W1, n)
  spmm1 = _make_spmm(n, 2 * n, 128, ep // NS, col_split=True)
  h1 = spmm1(sup1, src2, dst, w, zer)
  sup2 = _mm2(h1, W2p, n)
  spmm2 = _make_spmm(n, n, 128, ep // (NC * NS), col_split=False)
  p = spmm2(sup2, src, dst, w, zer)
  return _finish(p, n, ncls)
